# baseline (device time: 104709 ns/iter reference)
import jax
import jax.numpy as jnp
from jax import lax
from jax.experimental import pallas as pl
from jax.experimental.pallas import tpu as pltpu

N_DEV = 4
BS = 128


def kernel(x, w_mat):
    M, K = x.shape
    _, N = w_mat.shape
    Mc = M // N_DEV
    NB = N // BS

    def body(
        x_ref, w_ref, out_ref,
        xbuf_ref, wbf_ref,
        send_q_ref, send_s_ref, recv_q_ref, recv_s_ref,
        amax_send_ref, amax_recv_ref,
        xcopy_sems,
        q_send_sems, q_recv_sems, s_send_sems, s_recv_sems,
        amax_send_sems, amax_recv_sems,
    ):
        my = lax.axis_index("i")

        def x_copy(c, slot):
            return pltpu.make_async_copy(
                x_ref.at[pl.ds(c * Mc, Mc), :],
                xbuf_ref.at[slot],
                xcopy_sems.at[slot],
            )

        chunks = [lax.rem(my + t, N_DEV) for t in (1, 2, 3)] + [my]
        x_copy(chunks[0], 0).start()
        x_copy(chunks[1], 1).start()

        barrier = pltpu.get_barrier_semaphore()
        for t in range(1, N_DEV):
            pl.semaphore_signal(
                barrier, inc=1,
                device_id=(lax.rem(my + t, N_DEV),),
                device_id_type=pl.DeviceIdType.MESH,
            )
        pl.semaphore_wait(barrier, N_DEV - 1)

        wbf_ref[...] = w_ref[...].astype(jnp.bfloat16)

        def dot_chunk(k):
            slot = k % 2
            x_copy(chunks[k], slot).wait()
            p = jnp.dot(
                xbuf_ref[slot].astype(jnp.bfloat16), wbf_ref[...],
                preferred_element_type=jnp.float32,
            )
            if k + 2 < N_DEV:
                x_copy(chunks[k + 2], slot).start()
            return p

        for k in range(3):
            p = dot_chunk(k)
            pb = p.reshape(Mc, NB, BS)
            s = (
                jnp.maximum(
                    jnp.max(jnp.abs(pb), axis=2, keepdims=True) / 127.0,
                    1e-20,
                )
            ).astype(jnp.bfloat16)
            send_s_ref[k, :, :] = s.reshape(Mc, NB)
            send_q_ref[k, :, :] = (
                jnp.clip(jnp.round(pb / s.astype(jnp.float32)), -127.0, 127.0)
                .astype(jnp.int8)
                .reshape(Mc, N)
            )
            pltpu.make_async_remote_copy(
                src_ref=send_q_ref.at[k],
                dst_ref=recv_q_ref.at[k],
                send_sem=q_send_sems.at[k],
                recv_sem=q_recv_sems.at[k],
                device_id=(chunks[k],),
                device_id_type=pl.DeviceIdType.MESH,
            ).start()
            pltpu.make_async_remote_copy(
                src_ref=send_s_ref.at[k],
                dst_ref=recv_s_ref.at[k],
                send_sem=s_send_sems.at[k],
                recv_sem=s_recv_sems.at[k],
                device_id=(chunks[k],),
                device_id_type=pl.DeviceIdType.MESH,
            ).start()

        out_ref[...] = dot_chunk(3)

        def wait_recv(dst, sem):
            pltpu.make_async_remote_copy(
                src_ref=dst, dst_ref=dst, send_sem=sem, recv_sem=sem,
                device_id=(my,), device_id_type=pl.DeviceIdType.MESH,
            ).wait_recv()

        for j in range(3):
            wait_recv(recv_q_ref.at[j], q_recv_sems.at[j])
            wait_recv(recv_s_ref.at[j], s_recv_sems.at[j])

        def dq(j):
            return (
                recv_q_ref[j].reshape(Mc, NB, BS).astype(jnp.float32)
                * recv_s_ref[j].astype(jnp.float32).reshape(Mc, NB, 1)
            ).reshape(Mc, N)

        acc = out_ref[...] + ((dq(0) + dq(1)) + dq(2))
        out_ref[...] = acc

        amax = jnp.max(jnp.abs(acc))
        amax_send_ref[...] = jnp.full((8, 128), amax, jnp.float32)
        for t in range(1, N_DEV):
            tgt = lax.rem(my + t, N_DEV)
            pltpu.make_async_remote_copy(
                src_ref=amax_send_ref,
                dst_ref=amax_recv_ref.at[t - 1],
                send_sem=amax_send_sems.at[t - 1],
                recv_sem=amax_recv_sems.at[t - 1],
                device_id=(tgt,),
                device_id_type=pl.DeviceIdType.MESH,
            ).start()
        for j in range(N_DEV - 1):
            wait_recv(amax_recv_ref.at[j], amax_recv_sems.at[j])
            amax = jnp.maximum(amax, amax_recv_ref[j, 0, 0])

        scale = amax / 448.0
        q = jnp.clip(out_ref[...] / scale, -448.0, 448.0).astype(
            jnp.float8_e4m3fn
        )
        out_ref[...] = q.astype(jnp.float32) * scale

        def wait_send(src, sem):
            pltpu.make_async_remote_copy(
                src_ref=src, dst_ref=src, send_sem=sem, recv_sem=sem,
                device_id=(my,), device_id_type=pl.DeviceIdType.MESH,
            ).wait_send()

        for j in range(3):
            wait_send(send_q_ref.at[j], q_send_sems.at[j])
            wait_send(send_s_ref.at[j], s_send_sems.at[j])
            wait_send(amax_send_ref, amax_send_sems.at[j])

    return pl.pallas_call(
        body,
        out_shape=jax.ShapeDtypeStruct((Mc, N), jnp.float32),
        in_specs=[
            pl.BlockSpec(memory_space=pltpu.MemorySpace.HBM),
            pl.BlockSpec(memory_space=pltpu.VMEM),
        ],
        out_specs=pl.BlockSpec(memory_space=pltpu.VMEM),
        scratch_shapes=[
            pltpu.VMEM((2, Mc, K), jnp.float32),
            pltpu.VMEM((K, N), jnp.bfloat16),
            pltpu.VMEM((3, Mc, N), jnp.int8),
            pltpu.VMEM((3, Mc, NB), jnp.bfloat16),
            pltpu.VMEM((3, Mc, N), jnp.int8),
            pltpu.VMEM((3, Mc, NB), jnp.bfloat16),
            pltpu.VMEM((8, 128), jnp.float32),
            pltpu.VMEM((N_DEV - 1, 8, 128), jnp.float32),
            pltpu.SemaphoreType.DMA((2,)),
            pltpu.SemaphoreType.DMA((3,)),
            pltpu.SemaphoreType.DMA((3,)),
            pltpu.SemaphoreType.DMA((3,)),
            pltpu.SemaphoreType.DMA((3,)),
            pltpu.SemaphoreType.DMA((N_DEV - 1,)),
            pltpu.SemaphoreType.DMA((N_DEV - 1,)),
        ],
        compiler_params=pltpu.CompilerParams(
            vmem_limit_bytes=63 * 1024 * 1024,
            collective_id=0,
        ),
    )(x, w_mat)


# device time: 93037 ns/iter; 1.1255x vs baseline; 1.1255x over previous
import jax
import jax.numpy as jnp
from jax import lax
from jax.experimental import pallas as pl
from jax.experimental.pallas import tpu as pltpu

N_DEV = 4
BS = 128


def kernel(x, w_mat):
    M, K = x.shape
    _, N = w_mat.shape
    Mc = M // N_DEV
    NB = N // BS

    def body(
        x_ref, w_ref, out_ref,
        xbuf_ref, wbf_ref,
        send_q_ref, send_s_ref, recv_q_ref, recv_s_ref,
        amax_send_ref, amax_recv_ref,
        xcopy_sems,
        q_send_sems, q_recv_sems, s_send_sems, s_recv_sems,
        amax_send_sems, amax_recv_sems,
    ):
        my = lax.axis_index("i")

        def x_copy(c, slot):
            return pltpu.make_async_copy(
                x_ref.at[pl.ds(c * Mc, Mc), :],
                xbuf_ref.at[slot],
                xcopy_sems.at[slot],
            )

        chunks = [lax.rem(my + t, N_DEV) for t in (1, 2, 3)] + [my]
        x_copy(chunks[0], 0).start()
        x_copy(chunks[1], 1).start()

        barrier = pltpu.get_barrier_semaphore()
        for t in range(1, N_DEV):
            pl.semaphore_signal(
                barrier, inc=1,
                device_id=(lax.rem(my + t, N_DEV),),
                device_id_type=pl.DeviceIdType.MESH,
            )
        pl.semaphore_wait(barrier, N_DEV - 1)

        wbf_ref[...] = w_ref[...].astype(jnp.bfloat16)

        H = 2
        Mh = Mc // H

        for k in range(3):
            slot = k % 2
            x_copy(chunks[k], slot).wait()
            for h in range(H):
                rows = pl.ds(h * Mh, Mh)
                p = jnp.dot(
                    xbuf_ref[slot, h * Mh:(h + 1) * Mh, :].astype(
                        jnp.bfloat16
                    ),
                    wbf_ref[...],
                    preferred_element_type=jnp.float32,
                )
                pb = p.reshape(Mh, NB, BS)
                s = (
                    jnp.maximum(
                        jnp.max(jnp.abs(pb), axis=2, keepdims=True) / 127.0,
                        1e-20,
                    )
                ).astype(jnp.bfloat16)
                send_s_ref[k, rows, :] = s.reshape(Mh, NB)
                send_q_ref[k, rows, :] = (
                    jnp.clip(
                        jnp.round(pb / s.astype(jnp.float32)), -127.0, 127.0
                    )
                    .astype(jnp.int8)
                    .reshape(Mh, N)
                )
                i = k * H + h
                pltpu.make_async_remote_copy(
                    src_ref=send_q_ref.at[k, rows, :],
                    dst_ref=recv_q_ref.at[k, rows, :],
                    send_sem=q_send_sems.at[i],
                    recv_sem=q_recv_sems.at[i],
                    device_id=(chunks[k],),
                    device_id_type=pl.DeviceIdType.MESH,
                ).start()
                pltpu.make_async_remote_copy(
                    src_ref=send_s_ref.at[k, rows, :],
                    dst_ref=recv_s_ref.at[k, rows, :],
                    send_sem=s_send_sems.at[i],
                    recv_sem=s_recv_sems.at[i],
                    device_id=(chunks[k],),
                    device_id_type=pl.DeviceIdType.MESH,
                ).start()
            if k + 2 < N_DEV:
                x_copy(chunks[k + 2], slot).start()

        x_copy(chunks[3], 1).wait()
        out_ref[...] = jnp.dot(
            xbuf_ref[1].astype(jnp.bfloat16), wbf_ref[...],
            preferred_element_type=jnp.float32,
        )

        def wait_recv(dst, sem):
            pltpu.make_async_remote_copy(
                src_ref=dst, dst_ref=dst, send_sem=sem, recv_sem=sem,
                device_id=(my,), device_id_type=pl.DeviceIdType.MESH,
            ).wait_recv()

        for j in range(3):
            for h in range(H):
                i = j * H + h
                rows = pl.ds(h * Mh, Mh)
                wait_recv(recv_q_ref.at[j, rows, :], q_recv_sems.at[i])
                wait_recv(recv_s_ref.at[j, rows, :], s_recv_sems.at[i])
            dq = (
                recv_q_ref[j].reshape(Mc, NB, BS).astype(jnp.float32)
                * recv_s_ref[j].astype(jnp.float32).reshape(Mc, NB, 1)
            ).reshape(Mc, N)
            out_ref[...] = out_ref[...] + dq

        amax = jnp.max(jnp.abs(out_ref[...]))
        amax_send_ref[...] = jnp.full((8, 128), amax, jnp.float32)
        for t in range(1, N_DEV):
            tgt = lax.rem(my + t, N_DEV)
            pltpu.make_async_remote_copy(
                src_ref=amax_send_ref,
                dst_ref=amax_recv_ref.at[t - 1],
                send_sem=amax_send_sems.at[t - 1],
                recv_sem=amax_recv_sems.at[t - 1],
                device_id=(tgt,),
                device_id_type=pl.DeviceIdType.MESH,
            ).start()
        for j in range(N_DEV - 1):
            wait_recv(amax_recv_ref.at[j], amax_recv_sems.at[j])
            amax = jnp.maximum(amax, amax_recv_ref[j, 0, 0])

        scale = amax / 448.0
        q = jnp.clip(out_ref[...] / scale, -448.0, 448.0).astype(
            jnp.float8_e4m3fn
        )
        out_ref[...] = q.astype(jnp.float32) * scale

        def wait_send(src, sem):
            pltpu.make_async_remote_copy(
                src_ref=src, dst_ref=src, send_sem=sem, recv_sem=sem,
                device_id=(my,), device_id_type=pl.DeviceIdType.MESH,
            ).wait_send()

        for i in range(6):
            rows = pl.ds((i % H) * Mh, Mh)
            wait_send(send_q_ref.at[i // H, rows, :], q_send_sems.at[i])
            wait_send(send_s_ref.at[i // H, rows, :], s_send_sems.at[i])
        for j in range(3):
            wait_send(amax_send_ref, amax_send_sems.at[j])

    return pl.pallas_call(
        body,
        out_shape=jax.ShapeDtypeStruct((Mc, N), jnp.float32),
        in_specs=[
            pl.BlockSpec(memory_space=pltpu.MemorySpace.HBM),
            pl.BlockSpec(memory_space=pltpu.VMEM),
        ],
        out_specs=pl.BlockSpec(memory_space=pltpu.VMEM),
        scratch_shapes=[
            pltpu.VMEM((2, Mc, K), jnp.float32),
            pltpu.VMEM((K, N), jnp.bfloat16),
            pltpu.VMEM((3, Mc, N), jnp.int8),
            pltpu.VMEM((3, Mc, NB), jnp.bfloat16),
            pltpu.VMEM((3, Mc, N), jnp.int8),
            pltpu.VMEM((3, Mc, NB), jnp.bfloat16),
            pltpu.VMEM((8, 128), jnp.float32),
            pltpu.VMEM((N_DEV - 1, 8, 128), jnp.float32),
            pltpu.SemaphoreType.DMA((2,)),
            pltpu.SemaphoreType.DMA((6,)),
            pltpu.SemaphoreType.DMA((6,)),
            pltpu.SemaphoreType.DMA((6,)),
            pltpu.SemaphoreType.DMA((6,)),
            pltpu.SemaphoreType.DMA((N_DEV - 1,)),
            pltpu.SemaphoreType.DMA((N_DEV - 1,)),
        ],
        compiler_params=pltpu.CompilerParams(
            vmem_limit_bytes=63 * 1024 * 1024,
            collective_id=0,
        ),
    )(x, w_mat)


# device time: 88799 ns/iter; 1.1792x vs baseline; 1.0477x over previous
import jax
import jax.numpy as jnp
from jax import lax
from jax.experimental import pallas as pl
from jax.experimental.pallas import tpu as pltpu

N_DEV = 4
BS = 128


def kernel(x, w_mat):
    M, K = x.shape
    _, N = w_mat.shape
    Mc = M // N_DEV
    NB = N // BS

    def body(
        x_ref, w_ref, out_ref,
        xbuf_ref, wbf_ref,
        send_q_ref, send_s_ref, recv_q_ref, recv_s_ref,
        amax_send_ref, amax_recv_ref,
        xcopy_sems,
        q_send_sems, q_recv_sems, s_send_sems, s_recv_sems,
        amax_send_sems, amax_recv_sems,
    ):
        my = lax.axis_index("i")

        def x_copy(c, slot):
            return pltpu.make_async_copy(
                x_ref.at[pl.ds(c * Mc, Mc), :],
                xbuf_ref.at[slot],
                xcopy_sems.at[slot],
            )

        chunks = [lax.rem(my + t, N_DEV) for t in (1, 2, 3)] + [my]
        x_copy(chunks[0], 0).start()
        x_copy(chunks[1], 1).start()

        barrier = pltpu.get_barrier_semaphore()
        for t in range(1, N_DEV):
            pl.semaphore_signal(
                barrier, inc=1,
                device_id=(lax.rem(my + t, N_DEV),),
                device_id_type=pl.DeviceIdType.MESH,
            )
        wbf_ref[...] = w_ref[...].astype(jnp.bfloat16)
        pl.semaphore_wait(barrier, N_DEV - 1)

        H = 4
        Mh = Mc // H

        for k in range(3):
            slot = k % 2
            x_copy(chunks[k], slot).wait()
            for h in range(H):
                rows = pl.ds(h * Mh, Mh)
                p = jnp.dot(
                    xbuf_ref[slot, h * Mh:(h + 1) * Mh, :].astype(
                        jnp.bfloat16
                    ),
                    wbf_ref[...],
                    preferred_element_type=jnp.float32,
                )
                pb = p.reshape(Mh, NB, BS)
                s = (
                    jnp.maximum(
                        jnp.max(jnp.abs(pb), axis=2, keepdims=True) / 127.0,
                        1e-20,
                    )
                ).astype(jnp.bfloat16)
                send_s_ref[k, rows, :] = s.reshape(Mh, NB)
                send_q_ref[k, rows, :] = (
                    jnp.clip(
                        jnp.round(pb / s.astype(jnp.float32)), -127.0, 127.0
                    )
                    .astype(jnp.int8)
                    .reshape(Mh, N)
                )
                i = k * H + h
                pltpu.make_async_remote_copy(
                    src_ref=send_q_ref.at[k, rows, :],
                    dst_ref=recv_q_ref.at[k, rows, :],
                    send_sem=q_send_sems.at[i],
                    recv_sem=q_recv_sems.at[i],
                    device_id=(chunks[k],),
                    device_id_type=pl.DeviceIdType.MESH,
                ).start()
                pltpu.make_async_remote_copy(
                    src_ref=send_s_ref.at[k, rows, :],
                    dst_ref=recv_s_ref.at[k, rows, :],
                    send_sem=s_send_sems.at[i],
                    recv_sem=s_recv_sems.at[i],
                    device_id=(chunks[k],),
                    device_id_type=pl.DeviceIdType.MESH,
                ).start()
            if k + 2 < N_DEV:
                x_copy(chunks[k + 2], slot).start()

        x_copy(chunks[3], 1).wait()
        out_ref[...] = jnp.dot(
            xbuf_ref[1].astype(jnp.bfloat16), wbf_ref[...],
            preferred_element_type=jnp.float32,
        )

        def wait_recv(dst, sem):
            pltpu.make_async_remote_copy(
                src_ref=dst, dst_ref=dst, send_sem=sem, recv_sem=sem,
                device_id=(my,), device_id_type=pl.DeviceIdType.MESH,
            ).wait_recv()

        for j in range(3):
            for h in range(H):
                i = j * H + h
                rows = pl.ds(h * Mh, Mh)
                wait_recv(recv_q_ref.at[j, rows, :], q_recv_sems.at[i])
                wait_recv(recv_s_ref.at[j, rows, :], s_recv_sems.at[i])
                lo, hi = h * Mh, (h + 1) * Mh
                dq = (
                    recv_q_ref[j, lo:hi, :]
                    .reshape(Mh, NB, BS)
                    .astype(jnp.float32)
                    * recv_s_ref[j, lo:hi, :]
                    .astype(jnp.float32)
                    .reshape(Mh, NB, 1)
                ).reshape(Mh, N)
                out_ref[rows, :] = out_ref[lo:hi, :] + dq

        amax = jnp.max(jnp.abs(out_ref[...]))
        amax_send_ref[...] = jnp.full((8, 128), amax, jnp.float32)
        for t in range(1, N_DEV):
            tgt = lax.rem(my + t, N_DEV)
            pltpu.make_async_remote_copy(
                src_ref=amax_send_ref,
                dst_ref=amax_recv_ref.at[t - 1],
                send_sem=amax_send_sems.at[t - 1],
                recv_sem=amax_recv_sems.at[t - 1],
                device_id=(tgt,),
                device_id_type=pl.DeviceIdType.MESH,
            ).start()
        for j in range(N_DEV - 1):
            wait_recv(amax_recv_ref.at[j], amax_recv_sems.at[j])
            amax = jnp.maximum(amax, amax_recv_ref[j, 0, 0])

        scale = amax / 448.0
        q = jnp.clip(out_ref[...] / scale, -448.0, 448.0).astype(
            jnp.float8_e4m3fn
        )
        out_ref[...] = q.astype(jnp.float32) * scale

        def wait_send(src, sem):
            pltpu.make_async_remote_copy(
                src_ref=src, dst_ref=src, send_sem=sem, recv_sem=sem,
                device_id=(my,), device_id_type=pl.DeviceIdType.MESH,
            ).wait_send()

        for i in range(3 * H):
            rows = pl.ds((i % H) * Mh, Mh)
            wait_send(send_q_ref.at[i // H, rows, :], q_send_sems.at[i])
            wait_send(send_s_ref.at[i // H, rows, :], s_send_sems.at[i])
        for j in range(3):
            wait_send(amax_send_ref, amax_send_sems.at[j])

    return pl.pallas_call(
        body,
        out_shape=jax.ShapeDtypeStruct((Mc, N), jnp.float32),
        in_specs=[
            pl.BlockSpec(memory_space=pltpu.MemorySpace.HBM),
            pl.BlockSpec(memory_space=pltpu.VMEM),
        ],
        out_specs=pl.BlockSpec(memory_space=pltpu.VMEM),
        scratch_shapes=[
            pltpu.VMEM((2, Mc, K), jnp.float32),
            pltpu.VMEM((K, N), jnp.bfloat16),
            pltpu.VMEM((3, Mc, N), jnp.int8),
            pltpu.VMEM((3, Mc, NB), jnp.bfloat16),
            pltpu.VMEM((3, Mc, N), jnp.int8),
            pltpu.VMEM((3, Mc, NB), jnp.bfloat16),
            pltpu.VMEM((8, 128), jnp.float32),
            pltpu.VMEM((N_DEV - 1, 8, 128), jnp.float32),
            pltpu.SemaphoreType.DMA((2,)),
            pltpu.SemaphoreType.DMA((12,)),
            pltpu.SemaphoreType.DMA((12,)),
            pltpu.SemaphoreType.DMA((12,)),
            pltpu.SemaphoreType.DMA((12,)),
            pltpu.SemaphoreType.DMA((N_DEV - 1,)),
            pltpu.SemaphoreType.DMA((N_DEV - 1,)),
        ],
        compiler_params=pltpu.CompilerParams(
            vmem_limit_bytes=63 * 1024 * 1024,
            collective_id=0,
        ),
    )(x, w_mat)


# device time: 88385 ns/iter; 1.1847x vs baseline; 1.0047x over previous
import jax
import jax.numpy as jnp
from jax import lax
from jax.experimental import pallas as pl
from jax.experimental.pallas import tpu as pltpu

N_DEV = 4
BS = 128


def kernel(x, w_mat):
    M, K = x.shape
    _, N = w_mat.shape
    Mc = M // N_DEV
    NB = N // BS

    def body(
        x_ref, w_ref, out_ref,
        xbuf_ref, wbf_ref,
        send_q_ref, send_s_ref, recv_q_ref, recv_s_ref,
        amax_send_ref, amax_recv_ref,
        xcopy_sems,
        q_send_sems, q_recv_sems, s_send_sems, s_recv_sems,
        amax_send_sems, amax_recv_sems,
    ):
        my = lax.axis_index("i")

        def x_copy(c, slot):
            return pltpu.make_async_copy(
                x_ref.at[pl.ds(c * Mc, Mc), :],
                xbuf_ref.at[slot],
                xcopy_sems.at[slot],
            )

        chunks = [lax.rem(my + t, N_DEV) for t in (1, 2, 3)] + [my]
        x_copy(chunks[0], 0).start()
        x_copy(chunks[1], 1).start()

        barrier = pltpu.get_barrier_semaphore()
        for t in range(1, N_DEV):
            pl.semaphore_signal(
                barrier, inc=1,
                device_id=(lax.rem(my + t, N_DEV),),
                device_id_type=pl.DeviceIdType.MESH,
            )
        wbf_ref[...] = w_ref[...].astype(jnp.bfloat16)
        pl.semaphore_wait(barrier, N_DEV - 1)

        H = 4
        Mh = Mc // H

        for k in range(3):
            slot = k % 2
            x_copy(chunks[k], slot).wait()
            for h in range(H):
                rows = pl.ds(h * Mh, Mh)
                p = jnp.dot(
                    xbuf_ref[slot, h * Mh:(h + 1) * Mh, :].astype(
                        jnp.bfloat16
                    ),
                    wbf_ref[...],
                    preferred_element_type=jnp.float32,
                )
                pb = p.reshape(Mh, NB, BS)
                s = (
                    jnp.maximum(
                        jnp.max(jnp.abs(pb), axis=2, keepdims=True) / 127.0,
                        1e-20,
                    )
                ).astype(jnp.bfloat16)
                send_s_ref[k, rows, :] = s.reshape(Mh, NB)
                send_q_ref[k, rows, :] = (
                    jnp.clip(
                        jnp.round(pb / s.astype(jnp.float32)), -127.0, 127.0
                    )
                    .astype(jnp.int8)
                    .reshape(Mh, N)
                )
                i = k * H + h
                pltpu.make_async_remote_copy(
                    src_ref=send_q_ref.at[k, rows, :],
                    dst_ref=recv_q_ref.at[k, rows, :],
                    send_sem=q_send_sems.at[i],
                    recv_sem=q_recv_sems.at[i],
                    device_id=(chunks[k],),
                    device_id_type=pl.DeviceIdType.MESH,
                ).start()
                pltpu.make_async_remote_copy(
                    src_ref=send_s_ref.at[k, rows, :],
                    dst_ref=recv_s_ref.at[k, rows, :],
                    send_sem=s_send_sems.at[i],
                    recv_sem=s_recv_sems.at[i],
                    device_id=(chunks[k],),
                    device_id_type=pl.DeviceIdType.MESH,
                ).start()
            if k + 2 < N_DEV:
                x_copy(chunks[k + 2], slot).start()

        x_copy(chunks[3], 1).wait()
        out_ref[...] = jnp.dot(
            xbuf_ref[1].astype(jnp.bfloat16), wbf_ref[...],
            preferred_element_type=jnp.float32,
        )

        def wait_recv(dst, sem):
            pltpu.make_async_remote_copy(
                src_ref=dst, dst_ref=dst, send_sem=sem, recv_sem=sem,
                device_id=(my,), device_id_type=pl.DeviceIdType.MESH,
            ).wait_recv()

        amax = jnp.float32(0.0)
        for j in range(3):
            for h in range(H):
                i = j * H + h
                rows = pl.ds(h * Mh, Mh)
                wait_recv(recv_q_ref.at[j, rows, :], q_recv_sems.at[i])
                wait_recv(recv_s_ref.at[j, rows, :], s_recv_sems.at[i])
                lo, hi = h * Mh, (h + 1) * Mh
                dq = (
                    recv_q_ref[j, lo:hi, :]
                    .reshape(Mh, NB, BS)
                    .astype(jnp.float32)
                    * recv_s_ref[j, lo:hi, :]
                    .astype(jnp.float32)
                    .reshape(Mh, NB, 1)
                ).reshape(Mh, N)
                acc = out_ref[lo:hi, :] + dq
                out_ref[rows, :] = acc
                if j == 2:
                    amax = jnp.maximum(amax, jnp.max(jnp.abs(acc)))

        amax_send_ref[...] = jnp.full((8, 128), amax, jnp.float32)
        for t in range(1, N_DEV):
            tgt = lax.rem(my + t, N_DEV)
            pltpu.make_async_remote_copy(
                src_ref=amax_send_ref,
                dst_ref=amax_recv_ref.at[t - 1],
                send_sem=amax_send_sems.at[t - 1],
                recv_sem=amax_recv_sems.at[t - 1],
                device_id=(tgt,),
                device_id_type=pl.DeviceIdType.MESH,
            ).start()
        for j in range(N_DEV - 1):
            wait_recv(amax_recv_ref.at[j], amax_recv_sems.at[j])
            amax = jnp.maximum(amax, amax_recv_ref[j, 0, 0])

        scale = amax / 448.0
        q = jnp.clip(out_ref[...] / scale, -448.0, 448.0).astype(
            jnp.float8_e4m3fn
        )
        out_ref[...] = q.astype(jnp.float32) * scale

        def wait_send(src, sem):
            pltpu.make_async_remote_copy(
                src_ref=src, dst_ref=src, send_sem=sem, recv_sem=sem,
                device_id=(my,), device_id_type=pl.DeviceIdType.MESH,
            ).wait_send()

        for i in range(3 * H):
            rows = pl.ds((i % H) * Mh, Mh)
            wait_send(send_q_ref.at[i // H, rows, :], q_send_sems.at[i])
            wait_send(send_s_ref.at[i // H, rows, :], s_send_sems.at[i])
        for j in range(3):
            wait_send(amax_send_ref, amax_send_sems.at[j])

    return pl.pallas_call(
        body,
        out_shape=jax.ShapeDtypeStruct((Mc, N), jnp.float32),
        in_specs=[
            pl.BlockSpec(memory_space=pltpu.MemorySpace.HBM),
            pl.BlockSpec(memory_space=pltpu.VMEM),
        ],
        out_specs=pl.BlockSpec(memory_space=pltpu.VMEM),
        scratch_shapes=[
            pltpu.VMEM((2, Mc, K), jnp.float32),
            pltpu.VMEM((K, N), jnp.bfloat16),
            pltpu.VMEM((3, Mc, N), jnp.int8),
            pltpu.VMEM((3, Mc, NB), jnp.bfloat16),
            pltpu.VMEM((3, Mc, N), jnp.int8),
            pltpu.VMEM((3, Mc, NB), jnp.bfloat16),
            pltpu.VMEM((8, 128), jnp.float32),
            pltpu.VMEM((N_DEV - 1, 8, 128), jnp.float32),
            pltpu.SemaphoreType.DMA((2,)),
            pltpu.SemaphoreType.DMA((12,)),
            pltpu.SemaphoreType.DMA((12,)),
            pltpu.SemaphoreType.DMA((12,)),
            pltpu.SemaphoreType.DMA((12,)),
            pltpu.SemaphoreType.DMA((N_DEV - 1,)),
            pltpu.SemaphoreType.DMA((N_DEV - 1,)),
        ],
        compiler_params=pltpu.CompilerParams(
            vmem_limit_bytes=63 * 1024 * 1024,
            collective_id=0,
        ),
    )(x, w_mat)
